# baseline (speedup 1.0000x reference)
import functools

import jax
import jax.numpy as jnp
from jax import lax
from jax.experimental import pallas as pl
from jax.experimental.pallas import tpu as pltpu
from jax.experimental.pallas import tpu_sc as plsc

N = 10000
E = 320000
H = 128
G = 64
C = 10

NC = 2
NS = 16
NW = NC * NS
EB = 128
NB = 80
HB = NB // 2
EPAD = NW * NB * EB
NPAD = N + 8
CS = 80
NCH = N // CS
KCH = -(-NCH // NS)

_mesh = plsc.VectorSubcoreMesh(
    core_axis_name="c", subcore_axis_name="s", num_cores=NC, num_subcores=NS)


@functools.partial(
    pl.kernel,
    out_type=jax.ShapeDtypeStruct((NC * N, H), jnp.float32),
    mesh=_mesh,
    scratch_types=[
        pltpu.VMEM_SHARED((NPAD, H), jnp.float32),
        pltpu.VMEM((EB,), jnp.int32),
        pltpu.VMEM((EB,), jnp.int32),
        pltpu.VMEM((EB, H), jnp.float32),
        pltpu.SemaphoreType.DMA,
    ],
)
def _sc_agg(h_hbm, src_hbm, dst_hbm, out_hbm, agg_sh, src_v, dst_v, rows_v, sem):
    cid = lax.axis_index("c")
    sid = lax.axis_index("s")
    wid = sid * NC + cid

    def _zero_row(r, _):
        for c8 in range(H // 16):
            rows_v[r, pl.ds(c8 * 16, 16)] = jnp.zeros((16,), jnp.float32)
        return 0
    lax.fori_loop(0, CS, _zero_row, 0)
    for k in range(KCH):
        c = sid + k * NS

        @pl.when(c < NCH)
        def _z():
            pltpu.sync_copy(rows_v.at[pl.ds(0, CS)], agg_sh.at[pl.ds(c * CS, CS)])
    plsc.subcore_barrier()

    def _edge_block(k, _):
        base = (wid * NB + k) * EB
        pltpu.sync_copy(src_hbm.at[pl.ds(base, EB)], src_v)
        pltpu.sync_copy(dst_hbm.at[pl.ds(base, EB)], dst_v)
        pltpu.async_copy(h_hbm.at[src_v], rows_v, sem).wait()
        pltpu.sync_copy(rows_v, agg_sh.at[dst_v], add=True)
        return 0
    lax.fori_loop(0, NB, _edge_block, 0)
    plsc.subcore_barrier()

    for k in range(KCH):
        c = sid + k * NS

        @pl.when(c < NCH)
        def _wb():
            pltpu.sync_copy(agg_sh.at[pl.ds(c * CS, CS)], rows_v.at[pl.ds(0, CS)])
            pltpu.sync_copy(rows_v.at[pl.ds(0, CS)],
                            out_hbm.at[pl.ds(cid * N + c * CS, CS)])


BR = 1000
_GRID = N // BR


def _dot_t(a, w):
    return lax.dot_general(a, w, (((1,), (1,)), ((), ())),
                           preferred_element_type=jnp.float32,
                           precision=lax.Precision.HIGHEST)


def _layer_body(p0_ref, p1_ref, h_ref, wr_ref, br_ref, ws_ref, o_ref):
    agg = p0_ref[...] + p1_ref[...]
    out = _dot_t(agg, wr_ref[...]) + br_ref[...] + _dot_t(h_ref[...], ws_ref[...])
    o_ref[...] = jnp.maximum(out, 0.0)


def _tc_layer(p, h, wr, br, ws):
    return pl.pallas_call(
        _layer_body,
        grid=(_GRID,),
        in_specs=[
            pl.BlockSpec((BR, H), lambda i: (i, 0)),
            pl.BlockSpec((BR, H), lambda i: (i + _GRID, 0)),
            pl.BlockSpec((BR, H), lambda i: (i, 0)),
            pl.BlockSpec((H, H), lambda i: (0, 0)),
            pl.BlockSpec((1, H), lambda i: (0, 0)),
            pl.BlockSpec((H, H), lambda i: (0, 0)),
        ],
        out_specs=pl.BlockSpec((BR, H), lambda i: (i, 0)),
        out_shape=jax.ShapeDtypeStruct((N, H), jnp.float32),
    )(p, p, h, wr, br, ws)


def _final_body(p0_ref, p1_ref, h_ref, b_ref, wr_ref, br_ref, ws_ref,
                wc_ref, bc_ref, o_ref, acc_ref, cnt_ref):
    i = pl.program_id(0)

    @pl.when(i == 0)
    def _init():
        acc_ref[...] = jnp.zeros((G, H), jnp.float32)
        cnt_ref[...] = jnp.zeros((1, G), jnp.float32)

    agg = p0_ref[...] + p1_ref[...]
    h3 = _dot_t(agg, wr_ref[...]) + br_ref[...] + _dot_t(h_ref[...], ws_ref[...])

    seg = b_ref[0, 0, :]
    onehot = (seg[:, None] ==
              lax.broadcasted_iota(jnp.int32, (BR, G), 1)).astype(jnp.float32)
    acc_ref[...] += lax.dot_general(onehot, h3, (((0,), (0,)), ((), ())),
                                    preferred_element_type=jnp.float32,
                                    precision=lax.Precision.HIGHEST)
    cnt_ref[...] += jnp.sum(onehot, axis=0, keepdims=True)

    @pl.when(i == _GRID - 1)
    def _finish():
        cnt = cnt_ref[...]
        pooled = acc_ref[...] / jnp.where(cnt > 0.0, cnt, 1.0).reshape(G, 1)
        o_ref[...] = _dot_t(pooled, wc_ref[...]) + bc_ref[...]


def _tc_final(p, h, batch3, wr, br, ws, wc, bc):
    return pl.pallas_call(
        _final_body,
        grid=(_GRID,),
        in_specs=[
            pl.BlockSpec((BR, H), lambda i: (i, 0)),
            pl.BlockSpec((BR, H), lambda i: (i + _GRID, 0)),
            pl.BlockSpec((BR, H), lambda i: (i, 0)),
            pl.BlockSpec((1, 1, BR), lambda i: (i, 0, 0)),
            pl.BlockSpec((H, H), lambda i: (0, 0)),
            pl.BlockSpec((1, H), lambda i: (0, 0)),
            pl.BlockSpec((H, H), lambda i: (0, 0)),
            pl.BlockSpec((C, H), lambda i: (0, 0)),
            pl.BlockSpec((1, C), lambda i: (0, 0)),
        ],
        out_specs=pl.BlockSpec((G, C), lambda i: (0, 0)),
        out_shape=jax.ShapeDtypeStruct((G, C), jnp.float32),
        scratch_shapes=[
            pltpu.VMEM((G, H), jnp.float32),
            pltpu.VMEM((1, G), jnp.float32),
        ],
    )(p, p, h, batch3, wr, br, ws, wc, bc)


def kernel(x, edge_index, batch, W1r, b1r, W1s, W2r, b2r, W2s, W3r, b3r, W3s,
           Wc, bc):
    pad = EPAD - E
    src = jnp.concatenate([edge_index[0], jnp.zeros((pad,), jnp.int32)])
    dst = jnp.concatenate([edge_index[1], jnp.full((pad,), N, jnp.int32)])
    batch3 = batch.reshape(_GRID, 1, BR)

    p = _sc_agg(x, src, dst)
    h1 = _tc_layer(p, x, W1r, b1r.reshape(1, H), W1s)
    p = _sc_agg(h1, src, dst)
    h2 = _tc_layer(p, h1, W2r, b2r.reshape(1, H), W2s)
    p = _sc_agg(h2, src, dst)
    return _tc_final(p, h2, batch3, W3r, b3r.reshape(1, H), W3s, Wc,
                     bc.reshape(1, C))


# speedup vs baseline: 1.6275x; 1.6275x over previous
import functools

import jax
import jax.numpy as jnp
from jax import lax
from jax.experimental import pallas as pl
from jax.experimental.pallas import tpu as pltpu
from jax.experimental.pallas import tpu_sc as plsc

N = 10000
E = 320000
H = 128
G = 64
C = 10

NC = 2
NS = 16
NW = NC * NS
EB = 128
NB = 79
EPAD = NW * NB * EB
NDUM = 1024
NPAD = N + NDUM
CS = 80
NCH = N // CS
KCH = -(-NCH // NS)

_mesh = plsc.VectorSubcoreMesh(
    core_axis_name="c", subcore_axis_name="s", num_cores=NC, num_subcores=NS)


@functools.partial(
    pl.kernel,
    out_type=jax.ShapeDtypeStruct((NC * N, H), jnp.float32),
    mesh=_mesh,
    scratch_types=[
        pltpu.VMEM_SHARED((NPAD, H), jnp.float32),
        pltpu.VMEM((EB,), jnp.int32),
        pltpu.VMEM((EB,), jnp.int32),
        pltpu.VMEM((EB, H), jnp.float32),
        pltpu.SemaphoreType.DMA,
    ],
)
def _sc_agg(h_hbm, src_hbm, dst_hbm, out_hbm, agg_sh, src_v, dst_v, rows_v, sem):
    cid = lax.axis_index("c")
    sid = lax.axis_index("s")
    wid = sid * NC + cid

    def _zero_row(r, _):
        for c8 in range(H // 16):
            rows_v[r, pl.ds(c8 * 16, 16)] = jnp.zeros((16,), jnp.float32)
        return 0
    lax.fori_loop(0, CS, _zero_row, 0)
    for k in range(KCH):
        c = sid + k * NS

        @pl.when(c < NCH)
        def _z():
            pltpu.sync_copy(rows_v.at[pl.ds(0, CS)], agg_sh.at[pl.ds(c * CS, CS)])
    plsc.subcore_barrier()

    def _edge_block(k, _):
        base = (wid + k * NW) * EB
        pltpu.sync_copy(src_hbm.at[pl.ds(base, EB)], src_v)
        pltpu.sync_copy(dst_hbm.at[pl.ds(base, EB)], dst_v)
        pltpu.async_copy(h_hbm.at[src_v], rows_v, sem).wait()
        pltpu.sync_copy(rows_v, agg_sh.at[dst_v], add=True)
        return 0
    lax.fori_loop(0, NB, _edge_block, 0)
    plsc.subcore_barrier()

    for k in range(KCH):
        c = sid + k * NS

        @pl.when(c < NCH)
        def _wb():
            pltpu.sync_copy(agg_sh.at[pl.ds(c * CS, CS)], rows_v.at[pl.ds(0, CS)])
            pltpu.sync_copy(rows_v.at[pl.ds(0, CS)],
                            out_hbm.at[pl.ds(cid * N + c * CS, CS)])


BR = 1000
_GRID = N // BR


def _dot_t(a, w):
    return lax.dot_general(a, w, (((1,), (1,)), ((), ())),
                           preferred_element_type=jnp.float32,
                           precision=lax.Precision.HIGHEST)


def _layer_body(p0_ref, p1_ref, h_ref, wr_ref, br_ref, ws_ref, o_ref):
    agg = p0_ref[...] + p1_ref[...]
    out = _dot_t(agg, wr_ref[...]) + br_ref[...] + _dot_t(h_ref[...], ws_ref[...])
    o_ref[...] = jnp.maximum(out, 0.0)


def _tc_layer(p, h, wr, br, ws):
    return pl.pallas_call(
        _layer_body,
        grid=(_GRID,),
        in_specs=[
            pl.BlockSpec((BR, H), lambda i: (i, 0)),
            pl.BlockSpec((BR, H), lambda i: (i + _GRID, 0)),
            pl.BlockSpec((BR, H), lambda i: (i, 0)),
            pl.BlockSpec((H, H), lambda i: (0, 0)),
            pl.BlockSpec((1, H), lambda i: (0, 0)),
            pl.BlockSpec((H, H), lambda i: (0, 0)),
        ],
        out_specs=pl.BlockSpec((BR, H), lambda i: (i, 0)),
        out_shape=jax.ShapeDtypeStruct((N, H), jnp.float32),
    )(p, p, h, wr, br, ws)


def _final_body(p0_ref, p1_ref, h_ref, b_ref, wr_ref, br_ref, ws_ref,
                wc_ref, bc_ref, o_ref, acc_ref, cnt_ref):
    i = pl.program_id(0)

    @pl.when(i == 0)
    def _init():
        acc_ref[...] = jnp.zeros((G, H), jnp.float32)
        cnt_ref[...] = jnp.zeros((1, G), jnp.float32)

    agg = p0_ref[...] + p1_ref[...]
    h3 = _dot_t(agg, wr_ref[...]) + br_ref[...] + _dot_t(h_ref[...], ws_ref[...])

    seg = b_ref[0, 0, :]
    onehot = (seg[:, None] ==
              lax.broadcasted_iota(jnp.int32, (BR, G), 1)).astype(jnp.float32)
    acc_ref[...] += lax.dot_general(onehot, h3, (((0,), (0,)), ((), ())),
                                    preferred_element_type=jnp.float32,
                                    precision=lax.Precision.HIGHEST)
    cnt_ref[...] += jnp.sum(onehot, axis=0, keepdims=True)

    @pl.when(i == _GRID - 1)
    def _finish():
        cnt = cnt_ref[...]
        pooled = acc_ref[...] / jnp.where(cnt > 0.0, cnt, 1.0).reshape(G, 1)
        o_ref[...] = _dot_t(pooled, wc_ref[...]) + bc_ref[...]


def _tc_final(p, h, batch3, wr, br, ws, wc, bc):
    return pl.pallas_call(
        _final_body,
        grid=(_GRID,),
        in_specs=[
            pl.BlockSpec((BR, H), lambda i: (i, 0)),
            pl.BlockSpec((BR, H), lambda i: (i + _GRID, 0)),
            pl.BlockSpec((BR, H), lambda i: (i, 0)),
            pl.BlockSpec((1, 1, BR), lambda i: (i, 0, 0)),
            pl.BlockSpec((H, H), lambda i: (0, 0)),
            pl.BlockSpec((1, H), lambda i: (0, 0)),
            pl.BlockSpec((H, H), lambda i: (0, 0)),
            pl.BlockSpec((C, H), lambda i: (0, 0)),
            pl.BlockSpec((1, C), lambda i: (0, 0)),
        ],
        out_specs=pl.BlockSpec((G, C), lambda i: (0, 0)),
        out_shape=jax.ShapeDtypeStruct((G, C), jnp.float32),
        scratch_shapes=[
            pltpu.VMEM((G, H), jnp.float32),
            pltpu.VMEM((1, G), jnp.float32),
        ],
    )(p, p, h, batch3, wr, br, ws, wc, bc)


def kernel(x, edge_index, batch, W1r, b1r, W1s, W2r, b2r, W2s, W3r, b3r, W3s,
           Wc, bc):
    pad = EPAD - E
    src = jnp.concatenate([edge_index[0], jnp.zeros((pad,), jnp.int32)])
    dst = jnp.concatenate(
        [edge_index[1], N + (jnp.arange(pad, dtype=jnp.int32) % NDUM)])
    batch3 = batch.reshape(_GRID, 1, BR)

    p = _sc_agg(x, src, dst)
    h1 = _tc_layer(p, x, W1r, b1r.reshape(1, H), W1s)
    p = _sc_agg(h1, src, dst)
    h2 = _tc_layer(p, h1, W2r, b2r.reshape(1, H), W2s)
    p = _sc_agg(h2, src, dst)
    return _tc_final(p, h2, batch3, W3r, b3r.reshape(1, H), W3s, Wc,
                     bc.reshape(1, C))


# speedup vs baseline: 3.6004x; 2.2122x over previous
import functools

import jax
import jax.numpy as jnp
from jax import lax
from jax.experimental import pallas as pl
from jax.experimental.pallas import tpu as pltpu
from jax.experimental.pallas import tpu_sc as plsc

N = 10000
E = 320000
H = 128
G = 64
C = 10

NC = 2
NS = 16
NW = NC * NS
EB = 128
NBLK = E // EB
CS = 80
NCH = N // CS
KCH = -(-NCH // NS)

_mesh = plsc.VectorSubcoreMesh(
    core_axis_name="c", subcore_axis_name="s", num_cores=NC, num_subcores=NS)


@functools.partial(
    pl.kernel,
    out_type=jax.ShapeDtypeStruct((NC * N, H), jnp.float32),
    mesh=_mesh,
    scratch_types=[
        pltpu.VMEM_SHARED((N, H), jnp.float32),
        pltpu.VMEM((EB,), jnp.int32),
        pltpu.VMEM((EB,), jnp.int32),
        pltpu.VMEM((EB,), jnp.int32),
        pltpu.VMEM((EB,), jnp.int32),
        pltpu.VMEM((EB, H), jnp.float32),
        pltpu.VMEM((EB, H), jnp.float32),
        pltpu.SemaphoreType.DMA,
        pltpu.SemaphoreType.DMA,
    ],
)
def _sc_agg(h_hbm, src_hbm, dst_hbm, out_hbm, agg_sh, src_a, dst_a, src_b,
            dst_b, rows_a, rows_b, sga, sgb):
    cid = lax.axis_index("c")
    sid = lax.axis_index("s")
    wid = sid * NC + cid

    def _zero_row(r, _):
        for c8 in range(H // 16):
            rows_a[r, pl.ds(c8 * 16, 16)] = jnp.zeros((16,), jnp.float32)
        return 0
    lax.fori_loop(0, CS, _zero_row, 0)
    for k in range(KCH):
        c = sid + k * NS

        @pl.when(c < NCH)
        def _z():
            pltpu.sync_copy(rows_a.at[pl.ds(0, CS)], agg_sh.at[pl.ds(c * CS, CS)])
    plsc.subcore_barrier()

    nblk = jnp.where(wid < NBLK - (NBLK // NW) * NW, NBLK // NW + 1, NBLK // NW)

    def _load_and_gather(k, src_x, dst_x, rows_x, sg_x):
        base = (wid + k * NW) * EB
        pltpu.sync_copy(src_hbm.at[pl.ds(base, EB)], src_x)
        pltpu.sync_copy(dst_hbm.at[pl.ds(base, EB)], dst_x)
        pltpu.async_copy(h_hbm.at[src_x], rows_x, sg_x)

    def _step(k, src_p, dst_p, rows_p, sg_p, src_q, dst_q, rows_q, sg_q):
        @pl.when(k + 1 < nblk)
        def _prefetch():
            _load_and_gather(k + 1, src_q, dst_q, rows_q, sg_q)
        pltpu.make_async_copy(h_hbm.at[pl.ds(0, EB)], rows_p, sg_p).wait()
        pltpu.sync_copy(rows_p, agg_sh.at[dst_p], add=True)

    def _edge_block(k, _):
        @pl.when(k % 2 == 0)
        def _a():
            _step(k, src_a, dst_a, rows_a, sga, src_b, dst_b, rows_b, sgb)

        @pl.when(k % 2 == 1)
        def _b():
            _step(k, src_b, dst_b, rows_b, sgb, src_a, dst_a, rows_a, sga)
        return 0

    _load_and_gather(0, src_a, dst_a, rows_a, sga)
    lax.fori_loop(0, nblk, _edge_block, 0)
    plsc.subcore_barrier()

    for k in range(KCH):
        c = sid + k * NS

        @pl.when(c < NCH)
        def _wb():
            pltpu.sync_copy(agg_sh.at[pl.ds(c * CS, CS)], rows_a.at[pl.ds(0, CS)])
            pltpu.sync_copy(rows_a.at[pl.ds(0, CS)],
                            out_hbm.at[pl.ds(cid * N + c * CS, CS)])


BR = 1000
_GRID = N // BR


def _dot_t(a, w):
    return lax.dot_general(a, w, (((1,), (1,)), ((), ())),
                           preferred_element_type=jnp.float32,
                           precision=lax.Precision.HIGHEST)


def _layer_body(p0_ref, p1_ref, h_ref, wr_ref, br_ref, ws_ref, o_ref):
    agg = p0_ref[...] + p1_ref[...]
    out = _dot_t(agg, wr_ref[...]) + br_ref[...] + _dot_t(h_ref[...], ws_ref[...])
    o_ref[...] = jnp.maximum(out, 0.0)


def _tc_layer(p, h, wr, br, ws):
    return pl.pallas_call(
        _layer_body,
        grid=(_GRID,),
        in_specs=[
            pl.BlockSpec((BR, H), lambda i: (i, 0)),
            pl.BlockSpec((BR, H), lambda i: (i + _GRID, 0)),
            pl.BlockSpec((BR, H), lambda i: (i, 0)),
            pl.BlockSpec((H, H), lambda i: (0, 0)),
            pl.BlockSpec((1, H), lambda i: (0, 0)),
            pl.BlockSpec((H, H), lambda i: (0, 0)),
        ],
        out_specs=pl.BlockSpec((BR, H), lambda i: (i, 0)),
        out_shape=jax.ShapeDtypeStruct((N, H), jnp.float32),
    )(p, p, h, wr, br, ws)


def _final_body(p0_ref, p1_ref, h_ref, b_ref, wr_ref, br_ref, ws_ref,
                wc_ref, bc_ref, o_ref, acc_ref, cnt_ref):
    i = pl.program_id(0)

    @pl.when(i == 0)
    def _init():
        acc_ref[...] = jnp.zeros((G, H), jnp.float32)
        cnt_ref[...] = jnp.zeros((1, G), jnp.float32)

    agg = p0_ref[...] + p1_ref[...]
    h3 = _dot_t(agg, wr_ref[...]) + br_ref[...] + _dot_t(h_ref[...], ws_ref[...])

    seg = b_ref[0, 0, :]
    onehot = (seg[:, None] ==
              lax.broadcasted_iota(jnp.int32, (BR, G), 1)).astype(jnp.float32)
    acc_ref[...] += lax.dot_general(onehot, h3, (((0,), (0,)), ((), ())),
                                    preferred_element_type=jnp.float32,
                                    precision=lax.Precision.HIGHEST)
    cnt_ref[...] += jnp.sum(onehot, axis=0, keepdims=True)

    @pl.when(i == _GRID - 1)
    def _finish():
        cnt = cnt_ref[...]
        pooled = acc_ref[...] / jnp.where(cnt > 0.0, cnt, 1.0).reshape(G, 1)
        o_ref[...] = _dot_t(pooled, wc_ref[...]) + bc_ref[...]


def _tc_final(p, h, batch3, wr, br, ws, wc, bc):
    return pl.pallas_call(
        _final_body,
        grid=(_GRID,),
        in_specs=[
            pl.BlockSpec((BR, H), lambda i: (i, 0)),
            pl.BlockSpec((BR, H), lambda i: (i + _GRID, 0)),
            pl.BlockSpec((BR, H), lambda i: (i, 0)),
            pl.BlockSpec((1, 1, BR), lambda i: (i, 0, 0)),
            pl.BlockSpec((H, H), lambda i: (0, 0)),
            pl.BlockSpec((1, H), lambda i: (0, 0)),
            pl.BlockSpec((H, H), lambda i: (0, 0)),
            pl.BlockSpec((C, H), lambda i: (0, 0)),
            pl.BlockSpec((1, C), lambda i: (0, 0)),
        ],
        out_specs=pl.BlockSpec((G, C), lambda i: (0, 0)),
        out_shape=jax.ShapeDtypeStruct((G, C), jnp.float32),
        scratch_shapes=[
            pltpu.VMEM((G, H), jnp.float32),
            pltpu.VMEM((1, G), jnp.float32),
        ],
    )(p, p, h, batch3, wr, br, ws, wc, bc)


def kernel(x, edge_index, batch, W1r, b1r, W1s, W2r, b2r, W2s, W3r, b3r, W3s,
           Wc, bc):
    src = edge_index[0]
    dst = edge_index[1]
    batch3 = batch.reshape(_GRID, 1, BR)

    p = _sc_agg(x, src, dst)
    h1 = _tc_layer(p, x, W1r, b1r.reshape(1, H), W1s)
    p = _sc_agg(h1, src, dst)
    h2 = _tc_layer(p, h1, W2r, b2r.reshape(1, H), W2s)
    p = _sc_agg(h2, src, dst)
    return _tc_final(p, h2, batch3, W3r, b3r.reshape(1, H), W3s, Wc,
                     bc.reshape(1, C))


# speedup vs baseline: 3.6258x; 1.0071x over previous
import functools

import jax
import jax.numpy as jnp
from jax import lax
from jax.experimental import pallas as pl
from jax.experimental.pallas import tpu as pltpu
from jax.experimental.pallas import tpu_sc as plsc

N = 10000
E = 320000
H = 128
G = 64
C = 10

NC = 2
NS = 16
NW = NC * NS
EB = 128
NBLK = E // EB
CS = 80
NCH = N // CS
KCH = -(-NCH // NS)

_mesh = plsc.VectorSubcoreMesh(
    core_axis_name="c", subcore_axis_name="s", num_cores=NC, num_subcores=NS)


@functools.partial(
    pl.kernel,
    out_type=jax.ShapeDtypeStruct((NC * N, H), jnp.float32),
    mesh=_mesh,
    scratch_types=[
        pltpu.VMEM_SHARED((N, H), jnp.float32),
        pltpu.VMEM((EB,), jnp.int32),
        pltpu.VMEM((EB,), jnp.int32),
        pltpu.VMEM((EB,), jnp.int32),
        pltpu.VMEM((EB,), jnp.int32),
        pltpu.VMEM((EB, H), jnp.float32),
        pltpu.VMEM((EB, H), jnp.float32),
        pltpu.SemaphoreType.DMA,
        pltpu.SemaphoreType.DMA,
    ],
)
def _sc_agg(h_hbm, src_hbm, dst_hbm, out_hbm, agg_sh, src_a, dst_a, src_b,
            dst_b, rows_a, rows_b, sga, sgb):
    cid = lax.axis_index("c")
    sid = lax.axis_index("s")
    wid = sid * NC + cid

    def _zero_row(r, _):
        for c8 in range(H // 16):
            rows_a[r, pl.ds(c8 * 16, 16)] = jnp.zeros((16,), jnp.float32)
        return 0
    lax.fori_loop(0, CS, _zero_row, 0)
    for k in range(KCH):
        c = sid + k * NS

        @pl.when(c < NCH)
        def _z():
            pltpu.sync_copy(rows_a.at[pl.ds(0, CS)], agg_sh.at[pl.ds(c * CS, CS)])
    plsc.subcore_barrier()

    nblk = jnp.where(wid < NBLK - (NBLK // NW) * NW, NBLK // NW + 1, NBLK // NW)

    def _load_and_gather(k, src_x, dst_x, rows_x, sg_x):
        base = (wid + k * NW) * EB
        pltpu.sync_copy(src_hbm.at[pl.ds(base, EB)], src_x)
        pltpu.sync_copy(dst_hbm.at[pl.ds(base, EB)], dst_x)
        pltpu.async_copy(h_hbm.at[src_x], rows_x, sg_x)

    def _step(k, src_p, dst_p, rows_p, sg_p, src_q, dst_q, rows_q, sg_q):
        @pl.when(k + 1 < nblk)
        def _prefetch():
            _load_and_gather(k + 1, src_q, dst_q, rows_q, sg_q)
        pltpu.make_async_copy(h_hbm.at[pl.ds(0, EB)], rows_p, sg_p).wait()
        pltpu.sync_copy(rows_p, agg_sh.at[dst_p], add=True)

    def _edge_block(k, _):
        @pl.when(k % 2 == 0)
        def _a():
            _step(k, src_a, dst_a, rows_a, sga, src_b, dst_b, rows_b, sgb)

        @pl.when(k % 2 == 1)
        def _b():
            _step(k, src_b, dst_b, rows_b, sgb, src_a, dst_a, rows_a, sga)
        return 0

    _load_and_gather(0, src_a, dst_a, rows_a, sga)
    lax.fori_loop(0, nblk, _edge_block, 0)
    plsc.subcore_barrier()

    for k in range(KCH):
        c = sid + k * NS

        @pl.when(c < NCH)
        def _wb():
            pltpu.sync_copy(agg_sh.at[pl.ds(c * CS, CS)],
                            out_hbm.at[pl.ds(cid * N + c * CS, CS)])


BR = 1000
_GRID = N // BR


def _dot_t(a, w):
    return lax.dot_general(a, w, (((1,), (1,)), ((), ())),
                           preferred_element_type=jnp.float32,
                           precision=lax.Precision.HIGHEST)


def _layer_body(p0_ref, p1_ref, h_ref, wr_ref, br_ref, ws_ref, o_ref):
    agg = p0_ref[...] + p1_ref[...]
    out = _dot_t(agg, wr_ref[...]) + br_ref[...] + _dot_t(h_ref[...], ws_ref[...])
    o_ref[...] = jnp.maximum(out, 0.0)


def _tc_layer(p, h, wr, br, ws):
    return pl.pallas_call(
        _layer_body,
        grid=(_GRID,),
        in_specs=[
            pl.BlockSpec((BR, H), lambda i: (i, 0)),
            pl.BlockSpec((BR, H), lambda i: (i + _GRID, 0)),
            pl.BlockSpec((BR, H), lambda i: (i, 0)),
            pl.BlockSpec((H, H), lambda i: (0, 0)),
            pl.BlockSpec((1, H), lambda i: (0, 0)),
            pl.BlockSpec((H, H), lambda i: (0, 0)),
        ],
        out_specs=pl.BlockSpec((BR, H), lambda i: (i, 0)),
        out_shape=jax.ShapeDtypeStruct((N, H), jnp.float32),
    )(p, p, h, wr, br, ws)


def _final_body(p0_ref, p1_ref, h_ref, b_ref, wr_ref, br_ref, ws_ref,
                wc_ref, bc_ref, o_ref, acc_ref, cnt_ref):
    i = pl.program_id(0)

    @pl.when(i == 0)
    def _init():
        acc_ref[...] = jnp.zeros((G, H), jnp.float32)
        cnt_ref[...] = jnp.zeros((1, G), jnp.float32)

    agg = p0_ref[...] + p1_ref[...]
    h3 = _dot_t(agg, wr_ref[...]) + br_ref[...] + _dot_t(h_ref[...], ws_ref[...])

    seg = b_ref[0, 0, :]
    onehot = (seg[:, None] ==
              lax.broadcasted_iota(jnp.int32, (BR, G), 1)).astype(jnp.float32)
    acc_ref[...] += lax.dot_general(onehot, h3, (((0,), (0,)), ((), ())),
                                    preferred_element_type=jnp.float32,
                                    precision=lax.Precision.HIGHEST)
    cnt_ref[...] += jnp.sum(onehot, axis=0, keepdims=True)

    @pl.when(i == _GRID - 1)
    def _finish():
        cnt = cnt_ref[...]
        pooled = acc_ref[...] / jnp.where(cnt > 0.0, cnt, 1.0).reshape(G, 1)
        o_ref[...] = _dot_t(pooled, wc_ref[...]) + bc_ref[...]


def _tc_final(p, h, batch3, wr, br, ws, wc, bc):
    return pl.pallas_call(
        _final_body,
        grid=(_GRID,),
        in_specs=[
            pl.BlockSpec((BR, H), lambda i: (i, 0)),
            pl.BlockSpec((BR, H), lambda i: (i + _GRID, 0)),
            pl.BlockSpec((BR, H), lambda i: (i, 0)),
            pl.BlockSpec((1, 1, BR), lambda i: (i, 0, 0)),
            pl.BlockSpec((H, H), lambda i: (0, 0)),
            pl.BlockSpec((1, H), lambda i: (0, 0)),
            pl.BlockSpec((H, H), lambda i: (0, 0)),
            pl.BlockSpec((C, H), lambda i: (0, 0)),
            pl.BlockSpec((1, C), lambda i: (0, 0)),
        ],
        out_specs=pl.BlockSpec((G, C), lambda i: (0, 0)),
        out_shape=jax.ShapeDtypeStruct((G, C), jnp.float32),
        scratch_shapes=[
            pltpu.VMEM((G, H), jnp.float32),
            pltpu.VMEM((1, G), jnp.float32),
        ],
    )(p, p, h, batch3, wr, br, ws, wc, bc)


def kernel(x, edge_index, batch, W1r, b1r, W1s, W2r, b2r, W2s, W3r, b3r, W3s,
           Wc, bc):
    src = edge_index[0]
    dst = edge_index[1]
    batch3 = batch.reshape(_GRID, 1, BR)

    p = _sc_agg(x, src, dst)
    h1 = _tc_layer(p, x, W1r, b1r.reshape(1, H), W1s)
    p = _sc_agg(h1, src, dst)
    h2 = _tc_layer(p, h1, W2r, b2r.reshape(1, H), W2s)
    p = _sc_agg(h2, src, dst)
    return _tc_final(p, h2, batch3, W3r, b3r.reshape(1, H), W3s, Wc,
                     bc.reshape(1, C))


# speedup vs baseline: 4.3515x; 1.2001x over previous
import functools

import jax
import jax.numpy as jnp
from jax import lax
from jax.experimental import pallas as pl
from jax.experimental.pallas import tpu as pltpu
from jax.experimental.pallas import tpu_sc as plsc

N = 10000
E = 320000
H = 128
G = 64
C = 10

NC = 2
NS = 16
NW = NC * NS
EB = 128
NBLK = E // EB
CS = 80
NCH = N // CS
KCH = -(-NCH // NS)

_mesh = plsc.VectorSubcoreMesh(
    core_axis_name="c", subcore_axis_name="s", num_cores=NC, num_subcores=NS)


@functools.partial(
    pl.kernel,
    out_type=jax.ShapeDtypeStruct((NC * N, H), jnp.float32),
    mesh=_mesh,
    scratch_types=[
        pltpu.VMEM_SHARED((N, H), jnp.float32),
        pltpu.VMEM((EB,), jnp.int32),
        pltpu.VMEM((EB,), jnp.int32),
        pltpu.VMEM((EB,), jnp.int32),
        pltpu.VMEM((EB,), jnp.int32),
        pltpu.VMEM((EB,), jnp.int32),
        pltpu.VMEM((EB,), jnp.int32),
        pltpu.VMEM((EB, H), jnp.float32),
        pltpu.VMEM((EB, H), jnp.float32),
        pltpu.VMEM((EB, H), jnp.float32),
        pltpu.SemaphoreType.DMA,
        pltpu.SemaphoreType.DMA,
        pltpu.SemaphoreType.DMA,
        pltpu.SemaphoreType.DMA,
        pltpu.SemaphoreType.DMA,
        pltpu.SemaphoreType.DMA,
    ],
)
def _sc_agg(h_hbm, src_hbm, dst_hbm, out_hbm, agg_sh, src_a, dst_a, src_b,
            dst_b, src_c, dst_c, rows_a, rows_b, rows_c, sga, sgb, sgc,
            ssa, ssb, ssc):
    cid = lax.axis_index("c")
    sid = lax.axis_index("s")
    wid = sid * NC + cid

    def _zero_row(r, _):
        for c8 in range(H // 16):
            rows_a[r, pl.ds(c8 * 16, 16)] = jnp.zeros((16,), jnp.float32)
        return 0
    lax.fori_loop(0, CS, _zero_row, 0)
    for k in range(KCH):
        c = sid + k * NS

        @pl.when(c < NCH)
        def _z():
            pltpu.sync_copy(rows_a.at[pl.ds(0, CS)], agg_sh.at[pl.ds(c * CS, CS)])
    plsc.subcore_barrier()

    nblk = jnp.where(wid < NBLK - (NBLK // NW) * NW, NBLK // NW + 1, NBLK // NW)

    def _load_and_gather(k, src_x, dst_x, rows_x, sg_x):
        base = (wid + k * NW) * EB
        pltpu.sync_copy(src_hbm.at[pl.ds(base, EB)], src_x)
        pltpu.sync_copy(dst_hbm.at[pl.ds(base, EB)], dst_x)
        pltpu.async_copy(h_hbm.at[src_x], rows_x, sg_x)

    def _step(k, cur, nxt):
        src_p, dst_p, rows_p, sg_p, ss_p = cur
        src_q, dst_q, rows_q, sg_q, ss_q = nxt

        @pl.when(k + 1 < nblk)
        def _prefetch():
            @pl.when(k >= 2)
            def _drain_s():
                pltpu.make_async_copy(rows_q, agg_sh.at[dst_q], ss_q).wait()
            _load_and_gather(k + 1, src_q, dst_q, rows_q, sg_q)
        pltpu.make_async_copy(h_hbm.at[pl.ds(0, EB)], rows_p, sg_p).wait()
        pltpu.async_copy(rows_p, agg_sh.at[dst_p], ss_p, add=True)

    ring = ((src_a, dst_a, rows_a, sga, ssa),
            (src_b, dst_b, rows_b, sgb, ssb),
            (src_c, dst_c, rows_c, sgc, ssc))

    def _edge_block(k, _):
        for r in range(3):
            @pl.when(k % 3 == r)
            def _r(r=r):
                _step(k, ring[r], ring[(r + 1) % 3])
        return 0

    _load_and_gather(0, src_a, dst_a, rows_a, sga)
    lax.fori_loop(0, nblk, _edge_block, 0)
    for src_x, dst_x, rows_x, sg_x, ss_x in ring:
        pltpu.make_async_copy(rows_x, agg_sh.at[dst_x], ss_x).wait()
    plsc.subcore_barrier()

    for k in range(KCH):
        c = sid + k * NS

        @pl.when(c < NCH)
        def _wb():
            pltpu.sync_copy(agg_sh.at[pl.ds(c * CS, CS)],
                            out_hbm.at[pl.ds(cid * N + c * CS, CS)])


BR = 1000
_GRID = N // BR


def _dot_t(a, w):
    return lax.dot_general(a, w, (((1,), (1,)), ((), ())),
                           preferred_element_type=jnp.float32,
                           precision=lax.Precision.HIGHEST)


def _layer_body(p0_ref, p1_ref, h_ref, wr_ref, br_ref, ws_ref, o_ref):
    agg = p0_ref[...] + p1_ref[...]
    out = _dot_t(agg, wr_ref[...]) + br_ref[...] + _dot_t(h_ref[...], ws_ref[...])
    o_ref[...] = jnp.maximum(out, 0.0)


def _tc_layer(p, h, wr, br, ws):
    return pl.pallas_call(
        _layer_body,
        grid=(_GRID,),
        in_specs=[
            pl.BlockSpec((BR, H), lambda i: (i, 0)),
            pl.BlockSpec((BR, H), lambda i: (i + _GRID, 0)),
            pl.BlockSpec((BR, H), lambda i: (i, 0)),
            pl.BlockSpec((H, H), lambda i: (0, 0)),
            pl.BlockSpec((1, H), lambda i: (0, 0)),
            pl.BlockSpec((H, H), lambda i: (0, 0)),
        ],
        out_specs=pl.BlockSpec((BR, H), lambda i: (i, 0)),
        out_shape=jax.ShapeDtypeStruct((N, H), jnp.float32),
    )(p, p, h, wr, br, ws)


def _final_body(p0_ref, p1_ref, h_ref, b_ref, wr_ref, br_ref, ws_ref,
                wc_ref, bc_ref, o_ref, acc_ref, cnt_ref):
    i = pl.program_id(0)

    @pl.when(i == 0)
    def _init():
        acc_ref[...] = jnp.zeros((G, H), jnp.float32)
        cnt_ref[...] = jnp.zeros((1, G), jnp.float32)

    agg = p0_ref[...] + p1_ref[...]
    h3 = _dot_t(agg, wr_ref[...]) + br_ref[...] + _dot_t(h_ref[...], ws_ref[...])

    seg = b_ref[0, 0, :]
    onehot = (seg[:, None] ==
              lax.broadcasted_iota(jnp.int32, (BR, G), 1)).astype(jnp.float32)
    acc_ref[...] += lax.dot_general(onehot, h3, (((0,), (0,)), ((), ())),
                                    preferred_element_type=jnp.float32,
                                    precision=lax.Precision.HIGHEST)
    cnt_ref[...] += jnp.sum(onehot, axis=0, keepdims=True)

    @pl.when(i == _GRID - 1)
    def _finish():
        cnt = cnt_ref[...]
        pooled = acc_ref[...] / jnp.where(cnt > 0.0, cnt, 1.0).reshape(G, 1)
        o_ref[...] = _dot_t(pooled, wc_ref[...]) + bc_ref[...]


def _tc_final(p, h, batch3, wr, br, ws, wc, bc):
    return pl.pallas_call(
        _final_body,
        grid=(_GRID,),
        in_specs=[
            pl.BlockSpec((BR, H), lambda i: (i, 0)),
            pl.BlockSpec((BR, H), lambda i: (i + _GRID, 0)),
            pl.BlockSpec((BR, H), lambda i: (i, 0)),
            pl.BlockSpec((1, 1, BR), lambda i: (i, 0, 0)),
            pl.BlockSpec((H, H), lambda i: (0, 0)),
            pl.BlockSpec((1, H), lambda i: (0, 0)),
            pl.BlockSpec((H, H), lambda i: (0, 0)),
            pl.BlockSpec((C, H), lambda i: (0, 0)),
            pl.BlockSpec((1, C), lambda i: (0, 0)),
        ],
        out_specs=pl.BlockSpec((G, C), lambda i: (0, 0)),
        out_shape=jax.ShapeDtypeStruct((G, C), jnp.float32),
        scratch_shapes=[
            pltpu.VMEM((G, H), jnp.float32),
            pltpu.VMEM((1, G), jnp.float32),
        ],
    )(p, p, h, batch3, wr, br, ws, wc, bc)


def kernel(x, edge_index, batch, W1r, b1r, W1s, W2r, b2r, W2s, W3r, b3r, W3s,
           Wc, bc):
    src = edge_index[0]
    dst = edge_index[1]
    batch3 = batch.reshape(_GRID, 1, BR)

    p = _sc_agg(x, src, dst)
    h1 = _tc_layer(p, x, W1r, b1r.reshape(1, H), W1s)
    p = _sc_agg(h1, src, dst)
    h2 = _tc_layer(p, h1, W2r, b2r.reshape(1, H), W2s)
    p = _sc_agg(h2, src, dst)
    return _tc_final(p, h2, batch3, W3r, b3r.reshape(1, H), W3s, Wc,
                     bc.reshape(1, C))


# speedup vs baseline: 4.8625x; 1.1174x over previous
import functools

import jax
import jax.numpy as jnp
from jax import lax
from jax.experimental import pallas as pl
from jax.experimental.pallas import tpu as pltpu
from jax.experimental.pallas import tpu_sc as plsc

N = 10000
E = 320000
H = 128
G = 64
C = 10

NC = 2
NS = 16
NW = NC * NS
EB = 128
NBLK = E // EB
CS = 80
NCH = N // CS
KCH = -(-NCH // NS)

_mesh = plsc.VectorSubcoreMesh(
    core_axis_name="c", subcore_axis_name="s", num_cores=NC, num_subcores=NS)


@functools.partial(
    pl.kernel,
    out_type=jax.ShapeDtypeStruct((NC * N, H), jnp.float32),
    mesh=_mesh,
    scratch_types=[
        pltpu.VMEM_SHARED((N, H), jnp.float32),
        pltpu.VMEM((2, EB), jnp.int32),
        pltpu.VMEM((2, EB), jnp.int32),
        pltpu.VMEM((2, EB), jnp.int32),
        pltpu.VMEM((EB, H), jnp.float32),
        pltpu.VMEM((EB, H), jnp.float32),
        pltpu.VMEM((EB, H), jnp.float32),
        pltpu.SemaphoreType.DMA,
        pltpu.SemaphoreType.DMA,
        pltpu.SemaphoreType.DMA,
        pltpu.SemaphoreType.DMA,
        pltpu.SemaphoreType.DMA,
        pltpu.SemaphoreType.DMA,
    ],
)
def _sc_agg(h_hbm, eidx_hbm, out_hbm, agg_sh, idx_a, idx_b, idx_c,
            rows_a, rows_b, rows_c, sga, sgb, sgc, ssa, ssb, ssc):
    cid = lax.axis_index("c")
    sid = lax.axis_index("s")
    wid = sid * NC + cid

    def _zero_row(r, _):
        for c8 in range(H // 16):
            rows_a[r, pl.ds(c8 * 16, 16)] = jnp.zeros((16,), jnp.float32)
        return 0
    lax.fori_loop(0, CS, _zero_row, 0)
    for k in range(KCH):
        c = sid + k * NS

        @pl.when(c < NCH)
        def _z():
            pltpu.sync_copy(rows_a.at[pl.ds(0, CS)], agg_sh.at[pl.ds(c * CS, CS)])
    plsc.subcore_barrier()

    nblk = jnp.where(wid < NBLK - (NBLK // NW) * NW, NBLK // NW + 1, NBLK // NW)

    def _load_and_gather(k, idx_x, rows_x, sg_x):
        pltpu.sync_copy(eidx_hbm.at[wid + k * NW], idx_x)
        pltpu.async_copy(h_hbm.at[idx_x.at[0]], rows_x, sg_x)

    def _step(k, cur, nxt):
        idx_p, rows_p, sg_p, ss_p = cur
        idx_q, rows_q, sg_q, ss_q = nxt

        @pl.when(k + 1 < nblk)
        def _prefetch():
            @pl.when(k >= 2)
            def _drain_s():
                pltpu.make_async_copy(rows_q, agg_sh.at[idx_q.at[1]], ss_q).wait()
            _load_and_gather(k + 1, idx_q, rows_q, sg_q)
        pltpu.make_async_copy(h_hbm.at[pl.ds(0, EB)], rows_p, sg_p).wait()
        pltpu.async_copy(rows_p, agg_sh.at[idx_p.at[1]], ss_p, add=True)

    ring = ((idx_a, rows_a, sga, ssa),
            (idx_b, rows_b, sgb, ssb),
            (idx_c, rows_c, sgc, ssc))

    def _edge_block(k, _):
        for r in range(3):
            @pl.when(k % 3 == r)
            def _r(r=r):
                _step(k, ring[r], ring[(r + 1) % 3])
        return 0

    _load_and_gather(0, idx_a, rows_a, sga)
    lax.fori_loop(0, nblk, _edge_block, 0)
    for idx_x, rows_x, sg_x, ss_x in ring:
        pltpu.make_async_copy(rows_x, agg_sh.at[idx_x.at[1]], ss_x).wait()
    plsc.subcore_barrier()

    for k in range(KCH):
        c = sid + k * NS

        @pl.when(c < NCH)
        def _wb():
            pltpu.sync_copy(agg_sh.at[pl.ds(c * CS, CS)],
                            out_hbm.at[pl.ds(cid * N + c * CS, CS)])


BR = 1000
_GRID = N // BR


def _dot_t(a, w):
    return lax.dot_general(a, w, (((1,), (1,)), ((), ())),
                           preferred_element_type=jnp.float32,
                           precision=lax.Precision.HIGHEST)


def _layer_body(p0_ref, p1_ref, h_ref, wr_ref, br_ref, ws_ref, o_ref):
    agg = p0_ref[...] + p1_ref[...]
    out = _dot_t(agg, wr_ref[...]) + br_ref[...] + _dot_t(h_ref[...], ws_ref[...])
    o_ref[...] = jnp.maximum(out, 0.0)


def _tc_layer(p, h, wr, br, ws):
    return pl.pallas_call(
        _layer_body,
        grid=(_GRID,),
        in_specs=[
            pl.BlockSpec((BR, H), lambda i: (i, 0)),
            pl.BlockSpec((BR, H), lambda i: (i + _GRID, 0)),
            pl.BlockSpec((BR, H), lambda i: (i, 0)),
            pl.BlockSpec((H, H), lambda i: (0, 0)),
            pl.BlockSpec((1, H), lambda i: (0, 0)),
            pl.BlockSpec((H, H), lambda i: (0, 0)),
        ],
        out_specs=pl.BlockSpec((BR, H), lambda i: (i, 0)),
        out_shape=jax.ShapeDtypeStruct((N, H), jnp.float32),
    )(p, p, h, wr, br, ws)


def _final_body(p0_ref, p1_ref, h_ref, b_ref, wr_ref, br_ref, ws_ref,
                wc_ref, bc_ref, o_ref, acc_ref, cnt_ref):
    i = pl.program_id(0)

    @pl.when(i == 0)
    def _init():
        acc_ref[...] = jnp.zeros((G, H), jnp.float32)
        cnt_ref[...] = jnp.zeros((1, G), jnp.float32)

    agg = p0_ref[...] + p1_ref[...]
    h3 = _dot_t(agg, wr_ref[...]) + br_ref[...] + _dot_t(h_ref[...], ws_ref[...])

    seg = b_ref[0, 0, :]
    onehot = (seg[:, None] ==
              lax.broadcasted_iota(jnp.int32, (BR, G), 1)).astype(jnp.float32)
    acc_ref[...] += lax.dot_general(onehot, h3, (((0,), (0,)), ((), ())),
                                    preferred_element_type=jnp.float32,
                                    precision=lax.Precision.HIGHEST)
    cnt_ref[...] += jnp.sum(onehot, axis=0, keepdims=True)

    @pl.when(i == _GRID - 1)
    def _finish():
        cnt = cnt_ref[...]
        pooled = acc_ref[...] / jnp.where(cnt > 0.0, cnt, 1.0).reshape(G, 1)
        o_ref[...] = _dot_t(pooled, wc_ref[...]) + bc_ref[...]


def _tc_final(p, h, batch3, wr, br, ws, wc, bc):
    return pl.pallas_call(
        _final_body,
        grid=(_GRID,),
        in_specs=[
            pl.BlockSpec((BR, H), lambda i: (i, 0)),
            pl.BlockSpec((BR, H), lambda i: (i + _GRID, 0)),
            pl.BlockSpec((BR, H), lambda i: (i, 0)),
            pl.BlockSpec((1, 1, BR), lambda i: (i, 0, 0)),
            pl.BlockSpec((H, H), lambda i: (0, 0)),
            pl.BlockSpec((1, H), lambda i: (0, 0)),
            pl.BlockSpec((H, H), lambda i: (0, 0)),
            pl.BlockSpec((C, H), lambda i: (0, 0)),
            pl.BlockSpec((1, C), lambda i: (0, 0)),
        ],
        out_specs=pl.BlockSpec((G, C), lambda i: (0, 0)),
        out_shape=jax.ShapeDtypeStruct((G, C), jnp.float32),
        scratch_shapes=[
            pltpu.VMEM((G, H), jnp.float32),
            pltpu.VMEM((1, G), jnp.float32),
        ],
    )(p, p, h, batch3, wr, br, ws, wc, bc)


def kernel(x, edge_index, batch, W1r, b1r, W1s, W2r, b2r, W2s, W3r, b3r, W3s,
           Wc, bc):
    eidx = jnp.stack([edge_index[0].reshape(NBLK, EB),
                      edge_index[1].reshape(NBLK, EB)], axis=1)
    batch3 = batch.reshape(_GRID, 1, BR)

    p = _sc_agg(x, eidx)
    h1 = _tc_layer(p, x, W1r, b1r.reshape(1, H), W1s)
    p = _sc_agg(h1, eidx)
    h2 = _tc_layer(p, h1, W2r, b2r.reshape(1, H), W2s)
    p = _sc_agg(h2, eidx)
    return _tc_final(p, h2, batch3, W3r, b3r.reshape(1, H), W3s, Wc,
                     bc.reshape(1, C))
